# baseline (speedup 1.0000x reference)
import functools
import math

import jax
import jax.numpy as jnp
from jax import lax
from jax.experimental import pallas as pl
from jax.experimental.pallas import tpu as pltpu
from jax.experimental.pallas import tpu_sc as plsc

B = 4096
S = 200
EMB = 32
SCALE = math.sqrt(float(EMB))
NW = 32
BPW = B // NW
TOK_PW = BPW * S
SLAB = EMB * B
TILE = 1024
CH_STRIDE = 32 * TILE


def _emb_body(tok_hbm, table_hbm, out_hbm, tok_v, idx0, idx1, rows0, rows1,
              rpad, tbuf0, tbuf1, gsem0, gsem1, wsem0, wsem1):
    wid = lax.axis_index("s") * 2 + lax.axis_index("c")
    tok_base = wid * TOK_PW

    pltpu.sync_copy(tok_hbm.at[pl.ds(tok_base, TOK_PW)], tok_v)

    ci = lax.iota(jnp.int32, 16)
    jb = ci * S
    NR = 4 * BPW
    ROUNDS = S // 4

    def build_idx(rnd, idx_v):
        s0 = 4 * rnd
        for h in range(4):
            for m in range(8):
                t16 = plsc.load_gather(tok_v, [jb + (m * 16 * S + s0 + h)])
                idx_v[pl.ds(h * BPW + m * 16, 16)] = t16

    def transpose_rows(rows_v, tbuf):
        def rp_body(m, _):
            rb16 = 16 * m
            vals = []
            for j in range(16):
                vals.append(rows_v[rb16 + j, pl.ds(0, 16)])
                vals.append(rows_v[rb16 + j, pl.ds(16, 16)])
            for j in range(16):
                rpad[pl.ds((rb16 + j) * 33, 16)] = vals[2 * j]
                rpad[pl.ds((rb16 + j) * 33 + 16, 16)] = vals[2 * j + 1]
            return 0
        lax.fori_loop(0, NR // 16, rp_body, 0)

        def m_body(m, _):
            rib33 = (ci + 16 * m) * 33
            vals = [plsc.load_gather(rpad, [rib33 + c]) * SCALE
                    for c in range(EMB)]
            half = m // 8
            bl = 16 * m - 128 * half
            for c in range(EMB):
                tbuf[pl.ds(half * 4 * TILE
                           + (c // 8) * TILE + (c % 8) * 128 + bl, 16)] = vals[c]
            return 0
        lax.fori_loop(0, NR // 16, m_body, 0)

    def issue_writes(rnd, tbuf, wsem):
        for h in range(4):
            base = (4 * rnd + h) * SLAB + wid * TILE
            for ch in range(4):
                pltpu.async_copy(
                    tbuf.at[pl.ds(h * 4 * TILE + ch * TILE, TILE)],
                    out_hbm.at[pl.ds(base + ch * CH_STRIDE, TILE)], wsem)

    def drain_writes(wsem, tbuf):
        pltpu.make_async_copy(out_hbm.at[pl.ds(0, 16 * TILE)], tbuf, wsem).wait()

    build_idx(0, idx0)
    pltpu.async_copy(table_hbm.at[idx0], rows0, gsem0)

    def pair_body(t, _):
        r0 = 2 * t
        build_idx(r0 + 1, idx1)
        pltpu.async_copy(table_hbm.at[idx1], rows1, gsem1)
        pltpu.make_async_copy(table_hbm.at[idx0], rows0, gsem0).wait()

        transpose_rows(rows0, tbuf0)
        issue_writes(r0, tbuf0, wsem0)

        build_idx(jnp.minimum(r0 + 2, ROUNDS - 2), idx0)
        pltpu.async_copy(table_hbm.at[idx0], rows0, gsem0)
        pltpu.make_async_copy(table_hbm.at[idx1], rows1, gsem1).wait()

        transpose_rows(rows1, tbuf1)
        issue_writes(r0 + 1, tbuf1, wsem1)
        drain_writes(wsem0, tbuf0)
        drain_writes(wsem1, tbuf1)
        return 0

    lax.fori_loop(0, ROUNDS // 2, pair_body, 0)
    pltpu.make_async_copy(table_hbm.at[idx0], rows0, gsem0).wait()


V = 1000000
RBLK = 512
NBLK = V // RBLK
VTAIL = NBLK * RBLK


def _relayout_body(tab_t, out_hbm, vb0, vb1, vbp, tout0, tout1, vtail,
                   touttail, gsem0, gsem1, wsem0, wsem1):
    wid = lax.axis_index("s") * 2 + lax.axis_index("c")
    ci = lax.iota(jnp.int32, 16)

    def transpose_blk(vb, tout, nrows):
        stride = nrows + 1

        def rp_body(c, _):
            vals = [vb[c, pl.ds(16 * i, 16)] for i in range(nrows // 16)]
            for i in range(nrows // 16):
                vbp[pl.ds(c * stride + 16 * i, 16)] = vals[i]
            return 0
        lax.fori_loop(0, EMB, rp_body, 0)

        base0 = ci * stride
        base1 = (ci + 16) * stride

        def g_body(m, _):
            rb = 16 * m
            vals = []
            for j in range(16):
                vals.append(plsc.load_gather(vbp, [base0 + (rb + j)]))
                vals.append(plsc.load_gather(vbp, [base1 + (rb + j)]))
            for j in range(16):
                tout[pl.ds((rb + j) * EMB, 16)] = vals[2 * j]
                tout[pl.ds((rb + j) * EMB + 16, 16)] = vals[2 * j + 1]
            return 0
        lax.fori_loop(0, nrows // 16, g_body, 0)

    def blk_id(i):
        return jnp.minimum(wid + 32 * i, NBLK - 1)

    def fetch(i, vb, gsem):
        pltpu.async_copy(tab_t.at[:, pl.ds(blk_id(i) * RBLK, RBLK)], vb, gsem)

    def put(i, tout, wsem):
        pltpu.async_copy(tout, out_hbm.at[pl.ds(blk_id(i) * RBLK * EMB,
                                                RBLK * EMB)], wsem)

    def drain(wsem, tout):
        pltpu.make_async_copy(out_hbm.at[pl.ds(0, RBLK * EMB)], tout, wsem).wait()

    fetch(0, vb0, gsem0)
    niter = NBLK // 32 + 1

    def pair_body(t, _):
        i0 = 2 * t
        fetch(i0 + 1, vb1, gsem1)
        pltpu.make_async_copy(tab_t.at[:, pl.ds(0, RBLK)], vb0, gsem0).wait()
        transpose_blk(vb0, tout0, RBLK)
        put(i0, tout0, wsem0)
        fetch(i0 + 2, vb0, gsem0)
        pltpu.make_async_copy(tab_t.at[:, pl.ds(0, RBLK)], vb1, gsem1).wait()
        transpose_blk(vb1, tout1, RBLK)
        put(i0 + 1, tout1, wsem1)
        drain(wsem0, tout0)
        drain(wsem1, tout1)
        return 0

    lax.fori_loop(0, (niter + 1) // 2, pair_body, 0)
    pltpu.make_async_copy(tab_t.at[:, pl.ds(0, RBLK)], vb0, gsem0).wait()
    pltpu.sync_copy(tab_t.at[:, pl.ds(VTAIL, V - VTAIL)], vtail)
    transpose_blk(vtail, touttail, V - VTAIL)
    pltpu.sync_copy(touttail,
                    out_hbm.at[pl.ds(VTAIL * EMB, (V - VTAIL) * EMB)])


def _relayout_table(embedding_weight):
    mesh = plsc.VectorSubcoreMesh(core_axis_name="c", subcore_axis_name="s")
    run = functools.partial(
        pl.kernel,
        mesh=mesh,
        out_type=jax.ShapeDtypeStruct((V * EMB,), jnp.float32),
        scratch_types=[
            pltpu.VMEM((EMB, RBLK), jnp.float32),
            pltpu.VMEM((EMB, RBLK), jnp.float32),
            pltpu.VMEM((EMB * (RBLK + 1),), jnp.float32),
            pltpu.VMEM((RBLK * EMB,), jnp.float32),
            pltpu.VMEM((RBLK * EMB,), jnp.float32),
            pltpu.VMEM((EMB, V - VTAIL), jnp.float32),
            pltpu.VMEM(((V - VTAIL) * EMB,), jnp.float32),
            pltpu.SemaphoreType.DMA,
            pltpu.SemaphoreType.DMA,
            pltpu.SemaphoreType.DMA,
            pltpu.SemaphoreType.DMA,
        ],
        compiler_params=pltpu.CompilerParams(
            use_tc_tiling_on_sc=True, needs_layout_passes=False),
    )(_relayout_body)
    return run(embedding_weight.T)


@jax.jit
def kernel(tokens, embedding_weight):
    tok_flat = tokens.reshape(B * S).astype(jnp.int32)
    table_lin = _relayout_table(embedding_weight).reshape(V, EMB)
    mesh = plsc.VectorSubcoreMesh(core_axis_name="c", subcore_axis_name="s")
    run = functools.partial(
        pl.kernel,
        mesh=mesh,
        out_type=jax.ShapeDtypeStruct((B * S * EMB,), jnp.float32),
        scratch_types=[
            pltpu.VMEM((TOK_PW,), jnp.int32),
            pltpu.VMEM((4 * BPW,), jnp.int32),
            pltpu.VMEM((4 * BPW,), jnp.int32),
            pltpu.VMEM((4 * BPW, EMB), jnp.float32),
            pltpu.VMEM((4 * BPW, EMB), jnp.float32),
            pltpu.VMEM((4 * BPW * 33,), jnp.float32),
            pltpu.VMEM((16 * TILE,), jnp.float32),
            pltpu.VMEM((16 * TILE,), jnp.float32),
            pltpu.SemaphoreType.DMA,
            pltpu.SemaphoreType.DMA,
            pltpu.SemaphoreType.DMA,
            pltpu.SemaphoreType.DMA,
        ],
        compiler_params=pltpu.CompilerParams(
            use_tc_tiling_on_sc=False, needs_layout_passes=False),
    )(_emb_body)
    flat = run(tok_flat, table_lin)
    flat5 = flat.reshape(S, 4, B // 128, 8, 128)
    return flat5.transpose(2, 4, 0, 1, 3).reshape(B, S, EMB)


# speedup vs baseline: 1.0112x; 1.0112x over previous
import functools
import math

import jax
import jax.numpy as jnp
from jax import lax
from jax.experimental import pallas as pl
from jax.experimental.pallas import tpu as pltpu
from jax.experimental.pallas import tpu_sc as plsc

B = 4096
S = 200
EMB = 32
SCALE = math.sqrt(float(EMB))
NW = 32
BPW = B // NW
TOK_PW = BPW * S
SLAB = EMB * B
TILE = 1024
CH_STRIDE = 32 * TILE


def _emb_body(tok_hbm, table_hbm, out_hbm, tok_v, idx0, idx1, rows0, rows1,
              rpad, tbuf0, tbuf1, gsem0, gsem1, wsem0, wsem1):
    wid = lax.axis_index("s") * 2 + lax.axis_index("c")
    tok_base = wid * TOK_PW

    pltpu.sync_copy(tok_hbm.at[pl.ds(tok_base, TOK_PW)], tok_v)

    ci = lax.iota(jnp.int32, 16)
    jb = ci * S
    NR = 2 * BPW
    ROUNDS = S // 2

    def build_idx(rnd, idx_v):
        s0 = 2 * rnd
        for h in range(2):
            for m in range(8):
                t16 = plsc.load_gather(tok_v, [jb + (m * 16 * S + s0 + h)])
                idx_v[pl.ds(h * BPW + m * 16, 16)] = t16

    def transpose_rows(rows_v, tbuf):
        def rp_body(m, _):
            rb16 = 16 * m
            vals = []
            for j in range(16):
                vals.append(rows_v[rb16 + j, pl.ds(0, 16)])
                vals.append(rows_v[rb16 + j, pl.ds(16, 16)])
            for j in range(16):
                rpad[pl.ds((rb16 + j) * 33, 16)] = vals[2 * j]
                rpad[pl.ds((rb16 + j) * 33 + 16, 16)] = vals[2 * j + 1]
            return 0
        lax.fori_loop(0, NR // 16, rp_body, 0)

        def m_body(m, _):
            rib33 = (ci + 16 * m) * 33
            vals = [plsc.load_gather(rpad, [rib33 + c]) * SCALE
                    for c in range(EMB)]
            half = m // 8
            bl = 16 * m - 128 * half
            for c in range(EMB):
                tbuf[pl.ds(half * 4 * TILE
                           + (c // 8) * TILE + (c % 8) * 128 + bl, 16)] = vals[c]
            return 0
        lax.fori_loop(0, NR // 16, m_body, 0)

    def issue_writes(rnd, tbuf, wsem):
        for h in range(2):
            base = (2 * rnd + h) * SLAB + wid * TILE
            for ch in range(4):
                pltpu.async_copy(
                    tbuf.at[pl.ds(h * 4 * TILE + ch * TILE, TILE)],
                    out_hbm.at[pl.ds(base + ch * CH_STRIDE, TILE)], wsem)

    def drain_writes(wsem, tbuf):
        pltpu.make_async_copy(out_hbm.at[pl.ds(0, 8 * TILE)], tbuf, wsem).wait()

    build_idx(0, idx0)
    pltpu.async_copy(table_hbm.at[idx0], rows0, gsem0)

    def pair_body(t, _):
        r0 = 2 * t
        build_idx(r0 + 1, idx1)
        pltpu.async_copy(table_hbm.at[idx1], rows1, gsem1)
        pltpu.make_async_copy(table_hbm.at[idx0], rows0, gsem0).wait()

        transpose_rows(rows0, tbuf0)
        issue_writes(r0, tbuf0, wsem0)

        build_idx(jnp.minimum(r0 + 2, ROUNDS - 2), idx0)
        pltpu.async_copy(table_hbm.at[idx0], rows0, gsem0)
        pltpu.make_async_copy(table_hbm.at[idx1], rows1, gsem1).wait()

        transpose_rows(rows1, tbuf1)
        issue_writes(r0 + 1, tbuf1, wsem1)
        drain_writes(wsem0, tbuf0)
        drain_writes(wsem1, tbuf1)
        return 0

    lax.fori_loop(0, ROUNDS // 2, pair_body, 0)
    pltpu.make_async_copy(table_hbm.at[idx0], rows0, gsem0).wait()


V = 1000000
RBLK = 256
NBLK = V // RBLK
VTAIL = NBLK * RBLK


def _relayout_body(tab_t, out_hbm, vb0, vb1, vbp, tout0, tout1, vtail,
                   touttail, gsem0, gsem1, wsem0, wsem1):
    wid = lax.axis_index("s") * 2 + lax.axis_index("c")
    ci = lax.iota(jnp.int32, 16)

    def transpose_blk(vb, tout, nrows):
        stride = nrows + 1

        def rp_body(c, _):
            vals = [vb[c, pl.ds(16 * i, 16)] for i in range(nrows // 16)]
            for i in range(nrows // 16):
                vbp[pl.ds(c * stride + 16 * i, 16)] = vals[i]
            return 0
        lax.fori_loop(0, EMB, rp_body, 0)

        base0 = ci * stride
        base1 = (ci + 16) * stride

        def g_body(m, _):
            rb = 16 * m
            vals = []
            for j in range(16):
                vals.append(plsc.load_gather(vbp, [base0 + (rb + j)]))
                vals.append(plsc.load_gather(vbp, [base1 + (rb + j)]))
            for j in range(16):
                tout[pl.ds((rb + j) * EMB, 16)] = vals[2 * j]
                tout[pl.ds((rb + j) * EMB + 16, 16)] = vals[2 * j + 1]
            return 0
        lax.fori_loop(0, nrows // 16, g_body, 0)

    def blk_id(i):
        return jnp.minimum(wid + 32 * i, NBLK - 1)

    def fetch(i, vb, gsem):
        pltpu.async_copy(tab_t.at[:, pl.ds(blk_id(i) * RBLK, RBLK)], vb, gsem)

    def put(i, tout, wsem):
        pltpu.async_copy(tout, out_hbm.at[pl.ds(blk_id(i) * RBLK * EMB,
                                                RBLK * EMB)], wsem)

    def drain(wsem, tout):
        pltpu.make_async_copy(out_hbm.at[pl.ds(0, RBLK * EMB)], tout, wsem).wait()

    fetch(0, vb0, gsem0)
    niter = NBLK // 32 + 1

    def pair_body(t, _):
        i0 = 2 * t
        fetch(i0 + 1, vb1, gsem1)
        pltpu.make_async_copy(tab_t.at[:, pl.ds(0, RBLK)], vb0, gsem0).wait()
        transpose_blk(vb0, tout0, RBLK)
        put(i0, tout0, wsem0)
        fetch(i0 + 2, vb0, gsem0)
        pltpu.make_async_copy(tab_t.at[:, pl.ds(0, RBLK)], vb1, gsem1).wait()
        transpose_blk(vb1, tout1, RBLK)
        put(i0 + 1, tout1, wsem1)
        drain(wsem0, tout0)
        drain(wsem1, tout1)
        return 0

    lax.fori_loop(0, (niter + 1) // 2, pair_body, 0)
    pltpu.make_async_copy(tab_t.at[:, pl.ds(0, RBLK)], vb0, gsem0).wait()
    pltpu.sync_copy(tab_t.at[:, pl.ds(VTAIL, V - VTAIL)], vtail)
    transpose_blk(vtail, touttail, V - VTAIL)
    pltpu.sync_copy(touttail,
                    out_hbm.at[pl.ds(VTAIL * EMB, (V - VTAIL) * EMB)])


def _relayout_table(embedding_weight):
    mesh = plsc.VectorSubcoreMesh(core_axis_name="c", subcore_axis_name="s")
    run = functools.partial(
        pl.kernel,
        mesh=mesh,
        out_type=jax.ShapeDtypeStruct((V * EMB,), jnp.float32),
        scratch_types=[
            pltpu.VMEM((EMB, RBLK), jnp.float32),
            pltpu.VMEM((EMB, RBLK), jnp.float32),
            pltpu.VMEM((EMB * (RBLK + 1),), jnp.float32),
            pltpu.VMEM((RBLK * EMB,), jnp.float32),
            pltpu.VMEM((RBLK * EMB,), jnp.float32),
            pltpu.VMEM((EMB, V - VTAIL), jnp.float32),
            pltpu.VMEM(((V - VTAIL) * EMB,), jnp.float32),
            pltpu.SemaphoreType.DMA,
            pltpu.SemaphoreType.DMA,
            pltpu.SemaphoreType.DMA,
            pltpu.SemaphoreType.DMA,
        ],
        compiler_params=pltpu.CompilerParams(
            use_tc_tiling_on_sc=True, needs_layout_passes=False),
    )(_relayout_body)
    return run(embedding_weight.T)


@jax.jit
def kernel(tokens, embedding_weight):
    tok_flat = tokens.reshape(B * S).astype(jnp.int32)
    table_lin = _relayout_table(embedding_weight).reshape(V, EMB)
    mesh = plsc.VectorSubcoreMesh(core_axis_name="c", subcore_axis_name="s")
    run = functools.partial(
        pl.kernel,
        mesh=mesh,
        out_type=jax.ShapeDtypeStruct((B * S * EMB,), jnp.float32),
        scratch_types=[
            pltpu.VMEM((TOK_PW,), jnp.int32),
            pltpu.VMEM((2 * BPW,), jnp.int32),
            pltpu.VMEM((2 * BPW,), jnp.int32),
            pltpu.VMEM((2 * BPW, EMB), jnp.float32),
            pltpu.VMEM((2 * BPW, EMB), jnp.float32),
            pltpu.VMEM((2 * BPW * 33,), jnp.float32),
            pltpu.VMEM((8 * TILE,), jnp.float32),
            pltpu.VMEM((8 * TILE,), jnp.float32),
            pltpu.SemaphoreType.DMA,
            pltpu.SemaphoreType.DMA,
            pltpu.SemaphoreType.DMA,
            pltpu.SemaphoreType.DMA,
        ],
        compiler_params=pltpu.CompilerParams(
            use_tc_tiling_on_sc=False, needs_layout_passes=False),
    )(_emb_body)
    flat = run(tok_flat, table_lin)
    flat5 = flat.reshape(S, 4, B // 128, 8, 128)
    return flat5.transpose(2, 4, 0, 1, 3).reshape(B, S, EMB)
